# baseline (speedup 1.0000x reference)
import functools

import jax
import jax.numpy as jnp
from jax import lax
from jax.experimental import pallas as pl
from jax.experimental.pallas import tpu as pltpu
from jax.experimental.pallas import tpu_sc as plsc

N = 10000
E = 320000
D = 128
_DW = D // 2

_NSLAB = 2
_ES = E // _NSLAB

_NC = 2
_NS = 16
_NW = _NC * _NS
_EPW = _ES // _NW
_C = 40
_NCH = _EPW // _C
_NB = 4
_LK = 2


def _pre_body(x_ref, w1_ref, b1_ref, p_ref, q_ref):
    xv = x_ref[...]
    wa = w1_ref[:D, :] - w1_ref[D:, :]
    wb = w1_ref[D:, :]
    p = jnp.dot(xv, wa, preferred_element_type=jnp.float32) + b1_ref[...]
    q = jnp.dot(xv, wb, preferred_element_type=jnp.float32)
    p_ref[...] = p.astype(jnp.bfloat16)
    q_ref[...] = q.astype(jnp.bfloat16)


def _precompute(x, W1, b1):
    return pl.pallas_call(
        _pre_body,
        out_shape=(
            jax.ShapeDtypeStruct((N, D), jnp.bfloat16),
            jax.ShapeDtypeStruct((N, D), jnp.bfloat16),
        ),
    )(x, W1, b1.reshape(1, D))


def _pack_i32(a):
    return jax.lax.bitcast_convert_type(a.reshape(a.shape[0], _DW, 2),
                                        jnp.int32)


def _unpack_bf16(a):
    return jax.lax.bitcast_convert_type(a, jnp.bfloat16).reshape(a.shape[0], D)


def _gather_body(p_hbm, q_hbm, row_hbm, col_hbm, pg_hbm, qg_hbm,
                 idxr_v, idxc_v, prow_v, qrow_v, semp, semq, sop, soq):
    wid = lax.axis_index("s") * _NC + lax.axis_index("c")
    base = wid * _EPW

    pltpu.sync_copy(row_hbm.at[wid], idxr_v)
    pltpu.sync_copy(col_hbm.at[wid], idxc_v)

    def issue_gather(i, b):
        pltpu.async_copy(p_hbm.at[idxr_v.at[i]], prow_v[b], semp[b])
        pltpu.async_copy(q_hbm.at[idxc_v.at[i]], qrow_v[b], semq[b])

    def wait_gather(b):
        pltpu.make_async_copy(p_hbm.at[idxr_v.at[0]], prow_v[b], semp[b]).wait()
        pltpu.make_async_copy(q_hbm.at[idxc_v.at[0]], qrow_v[b], semq[b]).wait()

    def issue_store(i, b):
        dst = pl.ds(base + i * _C, _C)
        pltpu.async_copy(prow_v[b], pg_hbm.at[dst], sop[b])
        pltpu.async_copy(qrow_v[b], qg_hbm.at[dst], soq[b])

    def wait_store(b):
        pltpu.make_async_copy(prow_v[b], pg_hbm.at[pl.ds(base, _C)], sop[b]).wait()
        pltpu.make_async_copy(qrow_v[b], qg_hbm.at[pl.ds(base, _C)], soq[b]).wait()

    for i in range(_LK):
        issue_gather(i, i % _NB)

    def step(i, b):
        nxt = i + _LK
        nxt_b = (b + _LK) % _NB

        def prefetch():
            pl.when(i >= _NB - _LK)(lambda: wait_store(nxt_b))
            issue_gather(nxt, nxt_b)

        pl.when(nxt < _NCH)(prefetch)
        wait_gather(b)
        issue_store(i, b)

    def group(g, carry):
        for b in range(_NB):
            step(_NB * g + b, b)
        return carry

    lax.fori_loop(0, _NCH // _NB, group, 0)

    for k in range(_NCH - (_NCH // _NB) * _NB):
        i = (_NCH // _NB) * _NB + k
        wait_gather(k)
        issue_store(i, k)

    for b in range(_NB):
        wait_store(b)


def _gather_stage(Ppk, Qpk, row2, col2):
    mesh = plsc.VectorSubcoreMesh(core_axis_name="c", subcore_axis_name="s")
    fn = pl.kernel(
        _gather_body,
        out_type=(
            jax.ShapeDtypeStruct((_ES, _DW), jnp.int32),
            jax.ShapeDtypeStruct((_ES, _DW), jnp.int32),
        ),
        mesh=mesh,
        compiler_params=pltpu.CompilerParams(use_tc_tiling_on_sc=False),
        scratch_types=[
            pltpu.VMEM((_NCH, _C), jnp.int32),
            pltpu.VMEM((_NCH, _C), jnp.int32),
            [pltpu.VMEM((_C, _DW), jnp.int32)] * _NB,
            [pltpu.VMEM((_C, _DW), jnp.int32)] * _NB,
            [pltpu.SemaphoreType.DMA] * _NB,
            [pltpu.SemaphoreType.DMA] * _NB,
            [pltpu.SemaphoreType.DMA] * _NB,
            [pltpu.SemaphoreType.DMA] * _NB,
        ],
    )
    return fn(Ppk, Qpk, row2, col2)


_BE = 4000


def _mlp_body(pg_ref, qg_ref, w2_ref, b2_ref, o_ref):
    h = jnp.maximum(pg_ref[...] + qg_ref[...], jnp.bfloat16(0))
    o_ref[...] = jnp.maximum(
        jnp.dot(h, w2_ref[...].astype(jnp.bfloat16),
                preferred_element_type=jnp.float32)
        + b2_ref[...],
        0.0,
    )


def _mlp2(pg, qg, W2, b2):
    grid = (_ES // _BE,)
    return pl.pallas_call(
        _mlp_body,
        grid=grid,
        in_specs=[
            pl.BlockSpec((_BE, D), lambda i: (i, 0)),
            pl.BlockSpec((_BE, D), lambda i: (i, 0)),
            pl.BlockSpec((D, D), lambda i: (0, 0)),
            pl.BlockSpec((1, D), lambda i: (0, 0)),
        ],
        out_specs=pl.BlockSpec((_BE, D), lambda i: (i, 0)),
        out_shape=jax.ShapeDtypeStruct((_ES, D), jnp.float32),
    )(pg, qg, W2, b2.reshape(1, D))


@jax.jit
def kernel(x, edge_index, W1, b1, W2, b2):
    row = edge_index[0].astype(jnp.int32)
    col = edge_index[1].astype(jnp.int32)
    P, Q = _precompute(x, W1, b1)
    Ppk = _pack_i32(P)
    Qpk = _pack_i32(Q)
    outs = []
    for s in range(_NSLAB):
        row2 = lax.dynamic_slice_in_dim(row, s * _ES, _ES).reshape(_NW, _NCH, _C)
        col2 = lax.dynamic_slice_in_dim(col, s * _ES, _ES).reshape(_NW, _NCH, _C)
        pg, qg = _gather_stage(Ppk, Qpk, row2, col2)
        outs.append(_mlp2(_unpack_bf16(pg), _unpack_bf16(qg), W2, b2))
    return jnp.concatenate(outs, axis=0)


# speedup vs baseline: 3.2557x; 3.2557x over previous
import functools

import jax
import jax.numpy as jnp
from jax import lax
from jax.experimental import pallas as pl
from jax.experimental.pallas import tpu as pltpu
from jax.experimental.pallas import tpu_sc as plsc

N = 10000
E = 320000
D = 128
_DW = D // 2

_NC = 2
_NS = 16
_NW = _NC * _NS
_EPW = E // _NW
_C = 80
_NCH = _EPW // _C
_NB = 4
_LK = 2


def _pre_body(x_ref, w1_ref, b1_ref, p_ref, q_ref):
    xv = x_ref[...]
    wa = w1_ref[:D, :] - w1_ref[D:, :]
    wb = w1_ref[D:, :]
    p = jnp.dot(xv, wa, preferred_element_type=jnp.float32) + b1_ref[...]
    q = jnp.dot(xv, wb, preferred_element_type=jnp.float32)
    p_ref[...] = p.astype(jnp.bfloat16)
    q_ref[...] = q.astype(jnp.bfloat16)


def _precompute(x, W1, b1):
    return pl.pallas_call(
        _pre_body,
        out_shape=(
            jax.ShapeDtypeStruct((N, D), jnp.bfloat16),
            jax.ShapeDtypeStruct((N, D), jnp.bfloat16),
        ),
    )(x, W1, b1.reshape(1, D))


def _pack_i32(a):
    return jax.lax.bitcast_convert_type(a.reshape(a.shape[0], _DW, 2),
                                        jnp.int32)


def _gather_body(p_hbm, q_hbm, row_hbm, col_hbm, pg_hbm, qg_hbm,
                 idxr_v, idxc_v, prow_v, qrow_v, semp, semq, sop, soq):
    wid = lax.axis_index("s") * _NC + lax.axis_index("c")
    base = wid * _EPW

    pltpu.sync_copy(row_hbm.at[wid], idxr_v)
    pltpu.sync_copy(col_hbm.at[wid], idxc_v)

    def issue_gather(i, b):
        pltpu.async_copy(p_hbm.at[idxr_v.at[i]], prow_v[b], semp[b])
        pltpu.async_copy(q_hbm.at[idxc_v.at[i]], qrow_v[b], semq[b])

    def wait_gather(b):
        pltpu.make_async_copy(p_hbm.at[idxr_v.at[0]], prow_v[b], semp[b]).wait()
        pltpu.make_async_copy(q_hbm.at[idxc_v.at[0]], qrow_v[b], semq[b]).wait()

    def issue_store(i, b):
        dst = pl.ds(base + i * _C, _C)
        pltpu.async_copy(prow_v[b], pg_hbm.at[dst], sop[b])
        pltpu.async_copy(qrow_v[b], qg_hbm.at[dst], soq[b])

    def wait_store(b):
        pltpu.make_async_copy(prow_v[b], pg_hbm.at[pl.ds(base, _C)], sop[b]).wait()
        pltpu.make_async_copy(qrow_v[b], qg_hbm.at[pl.ds(base, _C)], soq[b]).wait()

    for i in range(_LK):
        issue_gather(i, i % _NB)

    def step(i, b):
        nxt = i + _LK
        nxt_b = (b + _LK) % _NB

        def prefetch():
            pl.when(i >= _NB - _LK)(lambda: wait_store(nxt_b))
            issue_gather(nxt, nxt_b)

        pl.when(nxt < _NCH)(prefetch)
        wait_gather(b)
        issue_store(i, b)

    def group(g, carry):
        for b in range(_NB):
            step(_NB * g + b, b)
        return carry

    lax.fori_loop(0, _NCH // _NB, group, 0)

    for k in range(_NCH - (_NCH // _NB) * _NB):
        i = (_NCH // _NB) * _NB + k
        wait_gather(k)
        issue_store(i, k)

    for b in range(_NB):
        wait_store(b)


def _gather_stage(Ppk, Qpk, row2, col2):
    mesh = plsc.VectorSubcoreMesh(core_axis_name="c", subcore_axis_name="s")
    fn = pl.kernel(
        _gather_body,
        out_type=(
            jax.ShapeDtypeStruct((E, _DW), jnp.int32),
            jax.ShapeDtypeStruct((E, _DW), jnp.int32),
        ),
        mesh=mesh,
        compiler_params=pltpu.CompilerParams(use_tc_tiling_on_sc=False),
        scratch_types=[
            pltpu.VMEM((_NCH, _C), jnp.int32),
            pltpu.VMEM((_NCH, _C), jnp.int32),
            [pltpu.VMEM((_C, _DW), jnp.int32)] * _NB,
            [pltpu.VMEM((_C, _DW), jnp.int32)] * _NB,
            [pltpu.SemaphoreType.DMA] * _NB,
            [pltpu.SemaphoreType.DMA] * _NB,
            [pltpu.SemaphoreType.DMA] * _NB,
            [pltpu.SemaphoreType.DMA] * _NB,
        ],
    )
    return fn(Ppk, Qpk, row2, col2)


_BE = 4000


def _mlp_body(pg_ref, qg_ref, w2e_ref, w2o_ref, b2_ref, o_ref):
    pw = pg_ref[...]
    qw = qg_ref[...]
    p_even = jax.lax.bitcast_convert_type(pw << 16, jnp.float32)
    p_odd = jax.lax.bitcast_convert_type(pw & (-65536), jnp.float32)
    q_even = jax.lax.bitcast_convert_type(qw << 16, jnp.float32)
    q_odd = jax.lax.bitcast_convert_type(qw & (-65536), jnp.float32)
    h_even = jnp.maximum(p_even + q_even, 0.0).astype(jnp.bfloat16)
    h_odd = jnp.maximum(p_odd + q_odd, 0.0).astype(jnp.bfloat16)
    acc = jnp.dot(h_even, w2e_ref[...].astype(jnp.bfloat16),
                  preferred_element_type=jnp.float32)
    acc += jnp.dot(h_odd, w2o_ref[...].astype(jnp.bfloat16),
                   preferred_element_type=jnp.float32)
    o_ref[...] = jnp.maximum(acc + b2_ref[...], 0.0)


def _mlp2(pg, qg, W2e, W2o, b2):
    grid = (E // _BE,)
    return pl.pallas_call(
        _mlp_body,
        grid=grid,
        in_specs=[
            pl.BlockSpec((_BE, _DW), lambda i: (i, 0)),
            pl.BlockSpec((_BE, _DW), lambda i: (i, 0)),
            pl.BlockSpec((_DW, D), lambda i: (0, 0)),
            pl.BlockSpec((_DW, D), lambda i: (0, 0)),
            pl.BlockSpec((1, D), lambda i: (0, 0)),
        ],
        out_specs=pl.BlockSpec((_BE, D), lambda i: (i, 0)),
        out_shape=jax.ShapeDtypeStruct((E, D), jnp.float32),
    )(pg, qg, W2e, W2o, b2.reshape(1, D))


@jax.jit
def kernel(x, edge_index, W1, b1, W2, b2):
    row = edge_index[0].astype(jnp.int32)
    col = edge_index[1].astype(jnp.int32)
    P, Q = _precompute(x, W1, b1)
    pg, qg = _gather_stage(_pack_i32(P), _pack_i32(Q),
                           row.reshape(_NW, _NCH, _C),
                           col.reshape(_NW, _NCH, _C))
    return _mlp2(pg, qg, W2[0::2], W2[1::2], b2)


# speedup vs baseline: 3.8795x; 1.1916x over previous
import functools

import jax
import jax.numpy as jnp
from jax import lax
from jax.experimental import pallas as pl
from jax.experimental.pallas import tpu as pltpu
from jax.experimental.pallas import tpu_sc as plsc

N = 10000
E = 320000
D = 128
_DW = D // 2

_NC = 2
_NS = 16
_NW = _NC * _NS
_EPW = E // _NW
_C = 80
_NCH = _EPW // _C
_NB = 4
_LK = 2


def _pre_body(x_ref, w1_ref, b1_ref, p_ref, q_ref):
    xv = x_ref[...]
    wa = w1_ref[:D, :] - w1_ref[D:, :]
    wb = w1_ref[D:, :]
    p = jnp.dot(xv, wa, preferred_element_type=jnp.float32) + b1_ref[...]
    q = jnp.dot(xv, wb, preferred_element_type=jnp.float32)
    p_ref[...] = p.astype(jnp.bfloat16)
    q_ref[...] = q.astype(jnp.bfloat16)


def _precompute(x, W1, b1):
    return pl.pallas_call(
        _pre_body,
        out_shape=(
            jax.ShapeDtypeStruct((N, D), jnp.bfloat16),
            jax.ShapeDtypeStruct((N, D), jnp.bfloat16),
        ),
    )(x, W1, b1.reshape(1, D))


def _pack_i32(a):
    return jax.lax.bitcast_convert_type(a.reshape(a.shape[0], _DW, 2),
                                        jnp.int32)


def _gather_body(p_hbm, q_hbm, row_hbm, col_hbm, g_hbm,
                 idxr_v, idxc_v, prow_v, qrow_v, s_v, semp, semq, so):
    wid = lax.axis_index("s") * _NC + lax.axis_index("c")
    base = wid * _EPW

    pltpu.sync_copy(row_hbm.at[wid], idxr_v)
    pltpu.sync_copy(col_hbm.at[wid], idxc_v)

    def issue_gather(i, b):
        pltpu.async_copy(p_hbm.at[idxr_v.at[i]], prow_v[b], semp[b])
        pltpu.async_copy(q_hbm.at[idxc_v.at[i]], qrow_v[b], semq[b])

    def wait_gather(b):
        pltpu.make_async_copy(p_hbm.at[idxr_v.at[0]], prow_v[b], semp[b]).wait()
        pltpu.make_async_copy(q_hbm.at[idxc_v.at[0]], qrow_v[b], semq[b]).wait()

    def repack(b):
        def rowfn(r, c2):
            for j in range(_DW // 16):
                sl = pl.ds(j * 16, 16)
                s_v[b][r, pl.ds(j * 16, 16)] = prow_v[b][r, sl]
                s_v[b][r, pl.ds(_DW + j * 16, 16)] = qrow_v[b][r, sl]
            return c2

        lax.fori_loop(0, _C, rowfn, 0, unroll=4)

    def issue_store(i, b):
        pltpu.async_copy(s_v[b], g_hbm.at[pl.ds(base + i * _C, _C)], so[b])

    def wait_store(b):
        pltpu.make_async_copy(s_v[b], g_hbm.at[pl.ds(base, _C)], so[b]).wait()

    for i in range(_LK):
        issue_gather(i, i % _NB)

    def step(i, b):
        nxt = i + _LK
        nxt_b = (b + _LK) % _NB
        pl.when(nxt < _NCH)(lambda: issue_gather(nxt, nxt_b))
        wait_gather(b)
        pl.when(i >= _NB)(lambda: wait_store(b))
        repack(b)
        issue_store(i, b)

    def group(g, carry):
        for b in range(_NB):
            step(_NB * g + b, b)
        return carry

    lax.fori_loop(0, _NCH // _NB, group, 0)

    for k in range(_NCH - (_NCH // _NB) * _NB):
        i = (_NCH // _NB) * _NB + k
        wait_gather(k)
        wait_store(k)
        repack(k)
        issue_store(i, k)

    for b in range(_NB):
        wait_store(b)


def _gather_stage(Ppk, Qpk, row2, col2):
    mesh = plsc.VectorSubcoreMesh(core_axis_name="c", subcore_axis_name="s")
    fn = pl.kernel(
        _gather_body,
        out_type=jax.ShapeDtypeStruct((E, D), jnp.int32),
        mesh=mesh,
        compiler_params=pltpu.CompilerParams(use_tc_tiling_on_sc=False),
        scratch_types=[
            pltpu.VMEM((_NCH, _C), jnp.int32),
            pltpu.VMEM((_NCH, _C), jnp.int32),
            [pltpu.VMEM((_C, _DW), jnp.int32)] * _NB,
            [pltpu.VMEM((_C, _DW), jnp.int32)] * _NB,
            [pltpu.VMEM((_C, D), jnp.int32)] * _NB,
            [pltpu.SemaphoreType.DMA] * _NB,
            [pltpu.SemaphoreType.DMA] * _NB,
            [pltpu.SemaphoreType.DMA] * _NB,
        ],
    )
    return fn(Ppk, Qpk, row2, col2)


_BE = 4000


def _mlp_body(g_ref, w2e_ref, w2o_ref, b2_ref, o_ref):
    gw = g_ref[...]
    pw = gw[:, :_DW]
    qw = gw[:, _DW:]
    p_even = jax.lax.bitcast_convert_type(pw << 16, jnp.float32)
    p_odd = jax.lax.bitcast_convert_type(pw & (-65536), jnp.float32)
    q_even = jax.lax.bitcast_convert_type(qw << 16, jnp.float32)
    q_odd = jax.lax.bitcast_convert_type(qw & (-65536), jnp.float32)
    h_even = jnp.maximum(p_even + q_even, 0.0).astype(jnp.bfloat16)
    h_odd = jnp.maximum(p_odd + q_odd, 0.0).astype(jnp.bfloat16)
    acc = jnp.dot(h_even, w2e_ref[...].astype(jnp.bfloat16),
                  preferred_element_type=jnp.float32)
    acc += jnp.dot(h_odd, w2o_ref[...].astype(jnp.bfloat16),
                   preferred_element_type=jnp.float32)
    o_ref[...] = jnp.maximum(acc + b2_ref[...], 0.0)


def _mlp2(g, W2e, W2o, b2):
    grid = (E // _BE,)
    return pl.pallas_call(
        _mlp_body,
        grid=grid,
        in_specs=[
            pl.BlockSpec((_BE, D), lambda i: (i, 0)),
            pl.BlockSpec((_DW, D), lambda i: (0, 0)),
            pl.BlockSpec((_DW, D), lambda i: (0, 0)),
            pl.BlockSpec((1, D), lambda i: (0, 0)),
        ],
        out_specs=pl.BlockSpec((_BE, D), lambda i: (i, 0)),
        out_shape=jax.ShapeDtypeStruct((E, D), jnp.float32),
    )(g, W2e, W2o, b2.reshape(1, D))


@jax.jit
def kernel(x, edge_index, W1, b1, W2, b2):
    row = edge_index[0].astype(jnp.int32)
    col = edge_index[1].astype(jnp.int32)
    P, Q = _precompute(x, W1, b1)
    g = _gather_stage(_pack_i32(P), _pack_i32(Q),
                      row.reshape(_NW, _NCH, _C),
                      col.reshape(_NW, _NCH, _C))
    return _mlp2(g, W2[0::2], W2[1::2], b2)


# speedup vs baseline: 5.6219x; 1.4491x over previous
import functools

import jax
import jax.numpy as jnp
from jax import lax
from jax.experimental import pallas as pl
from jax.experimental.pallas import tpu as pltpu
from jax.experimental.pallas import tpu_sc as plsc

N = 10000
E = 320000
D = 128
_DW = D // 2

_NC = 2
_NS = 16
_NW = _NC * _NS
_EPW = E // _NW
_C = 80
_NCH = _EPW // _C
_NB = 4
_LK = 2


def _pre_body(x_ref, w1_ref, b1_ref, p_ref, q_ref):
    xv = x_ref[...]
    wa = w1_ref[:D, :] - w1_ref[D:, :]
    wb = w1_ref[D:, :]
    p = jnp.dot(xv, wa, preferred_element_type=jnp.float32) + b1_ref[...]
    q = jnp.dot(xv, wb, preferred_element_type=jnp.float32)
    p_ref[...] = p.astype(jnp.bfloat16)
    q_ref[...] = q.astype(jnp.bfloat16)


def _precompute(x, W1, b1):
    return pl.pallas_call(
        _pre_body,
        out_shape=(
            jax.ShapeDtypeStruct((N, D), jnp.bfloat16),
            jax.ShapeDtypeStruct((N, D), jnp.bfloat16),
        ),
    )(x, W1, b1.reshape(1, D))


def _pack_i32(a):
    return jax.lax.bitcast_convert_type(a.reshape(a.shape[0], _DW, 2),
                                        jnp.int32)


def _gather_body(p_hbm, q_hbm, row_hbm, col_hbm, g_hbm,
                 idxr_v, idxc_v, prow_v, qrow_v, semp, semq, so):
    wid = lax.axis_index("s") * _NC + lax.axis_index("c")
    base = wid * _EPW

    pltpu.sync_copy(row_hbm.at[wid], idxr_v)
    pltpu.sync_copy(col_hbm.at[wid], idxc_v)

    def issue_gather(i, b):
        pltpu.async_copy(p_hbm.at[idxr_v.at[i]], prow_v[b], semp[b])
        pltpu.async_copy(q_hbm.at[idxc_v.at[i]], qrow_v[b], semq[b])

    def wait_gather(b):
        pltpu.make_async_copy(p_hbm.at[idxr_v.at[0]], prow_v[b], semp[b]).wait()
        pltpu.make_async_copy(q_hbm.at[idxc_v.at[0]], qrow_v[b], semq[b]).wait()

    def issue_store(i, b):
        rows = pl.ds(base + i * _C, _C)
        pltpu.async_copy(prow_v[b], g_hbm.at[rows, pl.ds(0, _DW)], so[b])
        pltpu.async_copy(qrow_v[b], g_hbm.at[rows, pl.ds(_DW, _DW)], so[b])

    def wait_store(b):
        rows = pl.ds(base, _C)
        pltpu.make_async_copy(prow_v[b], g_hbm.at[rows, pl.ds(0, _DW)],
                              so[b]).wait()
        pltpu.make_async_copy(qrow_v[b], g_hbm.at[rows, pl.ds(_DW, _DW)],
                              so[b]).wait()

    for i in range(_LK):
        issue_gather(i, i % _NB)

    def step(i, b):
        nxt = i + _LK
        nxt_b = (b + _LK) % _NB

        def prefetch():
            pl.when(i >= _NB - _LK)(lambda: wait_store(nxt_b))
            issue_gather(nxt, nxt_b)

        pl.when(nxt < _NCH)(prefetch)
        wait_gather(b)
        issue_store(i, b)

    def group(g, carry):
        for b in range(_NB):
            step(_NB * g + b, b)
        return carry

    lax.fori_loop(0, _NCH // _NB, group, 0)

    for k in range(_NCH - (_NCH // _NB) * _NB):
        i = (_NCH // _NB) * _NB + k
        wait_gather(k)
        issue_store(i, k)

    for b in range(_NB):
        wait_store(b)


def _gather_stage(Ppk, Qpk, row2, col2):
    mesh = plsc.VectorSubcoreMesh(core_axis_name="c", subcore_axis_name="s")
    fn = pl.kernel(
        _gather_body,
        out_type=jax.ShapeDtypeStruct((E, D), jnp.int32),
        mesh=mesh,
        compiler_params=pltpu.CompilerParams(use_tc_tiling_on_sc=False),
        scratch_types=[
            pltpu.VMEM((_NCH, _C), jnp.int32),
            pltpu.VMEM((_NCH, _C), jnp.int32),
            [pltpu.VMEM((_C, _DW), jnp.int32)] * _NB,
            [pltpu.VMEM((_C, _DW), jnp.int32)] * _NB,
            [pltpu.SemaphoreType.DMA] * _NB,
            [pltpu.SemaphoreType.DMA] * _NB,
            [pltpu.SemaphoreType.DMA] * _NB,
        ],
    )
    return fn(Ppk, Qpk, row2, col2)


_BE = 4000


def _mlp_body(g_ref, w2e_ref, w2o_ref, b2_ref, o_ref):
    gw = g_ref[...]
    pw = gw[:, :_DW]
    qw = gw[:, _DW:]
    p_even = jax.lax.bitcast_convert_type(pw << 16, jnp.float32)
    p_odd = jax.lax.bitcast_convert_type(pw & (-65536), jnp.float32)
    q_even = jax.lax.bitcast_convert_type(qw << 16, jnp.float32)
    q_odd = jax.lax.bitcast_convert_type(qw & (-65536), jnp.float32)
    h_even = jnp.maximum(p_even + q_even, 0.0).astype(jnp.bfloat16)
    h_odd = jnp.maximum(p_odd + q_odd, 0.0).astype(jnp.bfloat16)
    acc = jnp.dot(h_even, w2e_ref[...].astype(jnp.bfloat16),
                  preferred_element_type=jnp.float32)
    acc += jnp.dot(h_odd, w2o_ref[...].astype(jnp.bfloat16),
                   preferred_element_type=jnp.float32)
    o_ref[...] = jnp.maximum(acc + b2_ref[...], 0.0)


def _mlp2(g, W2e, W2o, b2):
    grid = (E // _BE,)
    return pl.pallas_call(
        _mlp_body,
        grid=grid,
        in_specs=[
            pl.BlockSpec((_BE, D), lambda i: (i, 0)),
            pl.BlockSpec((_DW, D), lambda i: (0, 0)),
            pl.BlockSpec((_DW, D), lambda i: (0, 0)),
            pl.BlockSpec((1, D), lambda i: (0, 0)),
        ],
        out_specs=pl.BlockSpec((_BE, D), lambda i: (i, 0)),
        out_shape=jax.ShapeDtypeStruct((E, D), jnp.float32),
    )(g, W2e, W2o, b2.reshape(1, D))


@jax.jit
def kernel(x, edge_index, W1, b1, W2, b2):
    row = edge_index[0].astype(jnp.int32)
    col = edge_index[1].astype(jnp.int32)
    P, Q = _precompute(x, W1, b1)
    g = _gather_stage(_pack_i32(P), _pack_i32(Q),
                      row.reshape(_NW, _NCH, _C),
                      col.reshape(_NW, _NCH, _C))
    return _mlp2(g, W2[0::2], W2[1::2], b2)


# speedup vs baseline: 5.6987x; 1.0137x over previous
import functools

import jax
import jax.numpy as jnp
from jax import lax
from jax.experimental import pallas as pl
from jax.experimental.pallas import tpu as pltpu
from jax.experimental.pallas import tpu_sc as plsc

N = 10000
E = 320000
D = 128
_DW = D // 2

_NC = 2
_NS = 16
_NW = _NC * _NS
_EPW = E // _NW
_NSLAB = 2
_ES = E // _NSLAB
_EPWS = _ES // _NW
_C = 40
_NCH = _EPWS // _C
_NB = 4
_LK = 2


def _pre_body(x_ref, w1_ref, b1_ref, p_ref, q_ref):
    xv = x_ref[...]
    wa = w1_ref[:D, :] - w1_ref[D:, :]
    wb = w1_ref[D:, :]
    p = jnp.dot(xv, wa, preferred_element_type=jnp.float32) + b1_ref[...]
    q = jnp.dot(xv, wb, preferred_element_type=jnp.float32)
    p_ref[...] = p.astype(jnp.bfloat16)
    q_ref[...] = q.astype(jnp.bfloat16)


def _precompute(x, W1, b1):
    return pl.pallas_call(
        _pre_body,
        out_shape=(
            jax.ShapeDtypeStruct((N, D), jnp.bfloat16),
            jax.ShapeDtypeStruct((N, D), jnp.bfloat16),
        ),
    )(x, W1, b1.reshape(1, D))


def _pack_i32(a):
    return jax.lax.bitcast_convert_type(a.reshape(a.shape[0], _DW, 2),
                                        jnp.int32)


def _gather_body(p_hbm, q_hbm, row_hbm, col_hbm, g_hbm,
                 idxr_v, idxc_v, prow_v, qrow_v, semp, semq, so):
    wid = lax.axis_index("s") * _NC + lax.axis_index("c")
    base = wid * _EPWS

    pltpu.sync_copy(row_hbm.at[wid], idxr_v)
    pltpu.sync_copy(col_hbm.at[wid], idxc_v)

    def issue_gather(i, b):
        pltpu.async_copy(p_hbm.at[idxr_v.at[i]], prow_v[b], semp[b])
        pltpu.async_copy(q_hbm.at[idxc_v.at[i]], qrow_v[b], semq[b])

    def wait_gather(b):
        pltpu.make_async_copy(p_hbm.at[idxr_v.at[0]], prow_v[b], semp[b]).wait()
        pltpu.make_async_copy(q_hbm.at[idxc_v.at[0]], qrow_v[b], semq[b]).wait()

    def issue_store(i, b):
        rows = pl.ds(base + i * _C, _C)
        pltpu.async_copy(prow_v[b], g_hbm.at[rows, pl.ds(0, _DW)], so[b])
        pltpu.async_copy(qrow_v[b], g_hbm.at[rows, pl.ds(_DW, _DW)], so[b])

    def wait_store(b):
        rows = pl.ds(base, _C)
        pltpu.make_async_copy(prow_v[b], g_hbm.at[rows, pl.ds(0, _DW)],
                              so[b]).wait()
        pltpu.make_async_copy(qrow_v[b], g_hbm.at[rows, pl.ds(_DW, _DW)],
                              so[b]).wait()

    for i in range(_LK):
        issue_gather(i, i % _NB)

    def step(i, b):
        nxt = i + _LK
        nxt_b = (b + _LK) % _NB

        def prefetch():
            pl.when(i >= _NB - _LK)(lambda: wait_store(nxt_b))
            issue_gather(nxt, nxt_b)

        pl.when(nxt < _NCH)(prefetch)
        wait_gather(b)
        issue_store(i, b)

    def group(g, carry):
        for b in range(_NB):
            step(_NB * g + b, b)
        return carry

    lax.fori_loop(0, _NCH // _NB, group, 0)

    for k in range(_NCH - (_NCH // _NB) * _NB):
        i = (_NCH // _NB) * _NB + k
        wait_gather(k)
        issue_store(i, k)

    for b in range(_NB):
        wait_store(b)


def _gather_stage(Ppk, Qpk, row2, col2):
    mesh = plsc.VectorSubcoreMesh(core_axis_name="c", subcore_axis_name="s")
    fn = pl.kernel(
        _gather_body,
        out_type=jax.ShapeDtypeStruct((_ES, D), jnp.int32),
        mesh=mesh,
        compiler_params=pltpu.CompilerParams(use_tc_tiling_on_sc=False),
        scratch_types=[
            pltpu.VMEM((_NCH, _C), jnp.int32),
            pltpu.VMEM((_NCH, _C), jnp.int32),
            [pltpu.VMEM((_C, _DW), jnp.int32)] * _NB,
            [pltpu.VMEM((_C, _DW), jnp.int32)] * _NB,
            [pltpu.SemaphoreType.DMA] * _NB,
            [pltpu.SemaphoreType.DMA] * _NB,
            [pltpu.SemaphoreType.DMA] * _NB,
        ],
    )
    return fn(Ppk, Qpk, row2, col2)


_BE = 4000


def _mlp_body(g_ref, w2e_ref, w2o_ref, b2_ref, o_ref):
    gw = g_ref[...]
    pw = gw[:, :_DW]
    qw = gw[:, _DW:]
    p_even = jax.lax.bitcast_convert_type(pw << 16, jnp.float32)
    p_odd = jax.lax.bitcast_convert_type(pw & (-65536), jnp.float32)
    q_even = jax.lax.bitcast_convert_type(qw << 16, jnp.float32)
    q_odd = jax.lax.bitcast_convert_type(qw & (-65536), jnp.float32)
    h_even = jnp.maximum(p_even + q_even, 0.0).astype(jnp.bfloat16)
    h_odd = jnp.maximum(p_odd + q_odd, 0.0).astype(jnp.bfloat16)
    acc = jnp.dot(h_even, w2e_ref[...].astype(jnp.bfloat16),
                  preferred_element_type=jnp.float32)
    acc += jnp.dot(h_odd, w2o_ref[...].astype(jnp.bfloat16),
                   preferred_element_type=jnp.float32)
    o_ref[...] = jnp.maximum(acc + b2_ref[...], 0.0)


def _mlp_body2(g_ref, w2e_ref, w2o_ref, b2_ref, oprev_ref, o_ref):
    _mlp_body(g_ref, w2e_ref, w2o_ref, b2_ref, o_ref)


_NBLK = _ES // _BE


def _mlp2_slab(g, W2e, W2o, b2, out_prev, blk0):
    if out_prev is None:
        return pl.pallas_call(
            _mlp_body,
            grid=(_NBLK,),
            in_specs=[
                pl.BlockSpec((_BE, D), lambda i: (i, 0)),
                pl.BlockSpec((_DW, D), lambda i: (0, 0)),
                pl.BlockSpec((_DW, D), lambda i: (0, 0)),
                pl.BlockSpec((1, D), lambda i: (0, 0)),
            ],
            out_specs=pl.BlockSpec((_BE, D), lambda i: (i + blk0, 0)),
            out_shape=jax.ShapeDtypeStruct((E, D), jnp.float32),
        )(g, W2e, W2o, b2.reshape(1, D))
    return pl.pallas_call(
        _mlp_body2,
        grid=(_NBLK,),
        in_specs=[
            pl.BlockSpec((_BE, D), lambda i: (i, 0)),
            pl.BlockSpec((_DW, D), lambda i: (0, 0)),
            pl.BlockSpec((_DW, D), lambda i: (0, 0)),
            pl.BlockSpec((1, D), lambda i: (0, 0)),
            pl.BlockSpec(memory_space=pl.ANY),
        ],
        out_specs=pl.BlockSpec((_BE, D), lambda i: (i + blk0, 0)),
        out_shape=jax.ShapeDtypeStruct((E, D), jnp.float32),
        input_output_aliases={4: 0},
    )(g, W2e, W2o, b2.reshape(1, D), out_prev)


@jax.jit
def kernel(x, edge_index, W1, b1, W2, b2):
    row = edge_index[0].astype(jnp.int32)
    col = edge_index[1].astype(jnp.int32)
    P, Q = _precompute(x, W1, b1)
    Ppk = _pack_i32(P)
    Qpk = _pack_i32(Q)
    W2e = W2[0::2]
    W2o = W2[1::2]
    gs = []
    for sidx in range(_NSLAB):
        r2 = row[sidx * _ES:(sidx + 1) * _ES].reshape(_NW, _NCH, _C)
        c2 = col[sidx * _ES:(sidx + 1) * _ES].reshape(_NW, _NCH, _C)
        gs.append(_gather_stage(Ppk, Qpk, r2, c2))
    out = None
    for sidx in range(_NSLAB):
        out = _mlp2_slab(gs[sidx], W2e, W2o, b2, out, sidx * _NBLK)
    return out
